# baseline (speedup 1.0000x reference)
import functools

import jax
import jax.numpy as jnp
from jax import lax
from jax.experimental import pallas as pl
from jax.experimental.pallas import tpu as pltpu
from jax.experimental.pallas import tpu_sc as plsc

B = 16
ROWS = 896
COLS = 5313
BIN0 = 416
NBIN = 64
HBIN = 32
NTRK = 32
TSTRIDE = 100
CW = 3200
OUT_N = B * NTRK

_GDN = lax.GatherDimensionNumbers(
    offset_dims=(), collapsed_slice_dims=(0,), start_index_map=(0,)
)


def _lane_bcast(v, p):
    idx = jnp.full((16, 1), p, dtype=jnp.int32)
    return lax.gather(
        v, idx, _GDN, (1,), mode=lax.GatherScatterMode.PROMISE_IN_BOUNDS
    )


def _sc_score(x):
    mesh = plsc.VectorSubcoreMesh(core_axis_name="c", subcore_axis_name="s")

    @functools.partial(
        pl.kernel,
        mesh=mesh,
        out_type=(
            jax.ShapeDtypeStruct((OUT_N,), jnp.float32),
            jax.ShapeDtypeStruct((2 * OUT_N,), jnp.float32),
        ),
        scratch_types=[
            pltpu.VMEM((HBIN, CW), jnp.float32),
            pltpu.VMEM((NTRK,), jnp.float32),
            pltpu.VMEM((NTRK,), jnp.float32),
            pltpu.SemaphoreType.DMA,
        ],
        compiler_params=pltpu.CompilerParams(use_tc_tiling_on_sc=True),
    )
    def k(in_hbm, out_hbm, stage_hbm, buf_v, mine_v, other_v, sem):
        c = lax.axis_index("c")
        s = lax.axis_index("s")
        w = c * 16 + s
        b = w // 2
        h = w % 2
        pltpu.async_copy(
            in_hbm.at[b, pl.ds(BIN0 + h * HBIN, HBIN), pl.ds(0, CW)],
            buf_v,
            sem,
        ).wait()
        lane = lax.iota(jnp.int32, 16)
        for r in range(2):
            res = jnp.zeros((16,), jnp.float32)
            for l in range(16):
                q = (r * 16 + l) * TSTRIDE
                a = (q // 16) * 16
                acc = buf_v[0, pl.ds(a, 16)]
                for i in range(1, HBIN):
                    acc = acc + buf_v[i, pl.ds(a, 16)]
                res = jnp.where(lane == l, _lane_bcast(acc, q - a), res)
            mine_v[pl.ds(r * 16, 16)] = res
        pltpu.sync_copy(mine_v, stage_hbm.at[pl.ds(w * NTRK, NTRK)])
        plsc.subcore_barrier()

        @pl.when(h == 0)
        def _():
            pltpu.sync_copy(stage_hbm.at[pl.ds((w + 1) * NTRK, NTRK)], other_v)
            for r in range(2):
                mine_v[pl.ds(r * 16, 16)] = (
                    mine_v[pl.ds(r * 16, 16)] + other_v[pl.ds(r * 16, 16)]
                )
            pltpu.sync_copy(mine_v, out_hbm.at[pl.ds(b * NTRK, NTRK)])

    return k(x)[0]


def kernel(inputs):
    out = _sc_score(inputs)
    return out.reshape(B, NTRK)


# speedup vs baseline: 13.8832x; 13.8832x over previous
import functools

import jax
import jax.numpy as jnp
from jax import lax
from jax.experimental import pallas as pl
from jax.experimental.pallas import tpu as pltpu
from jax.experimental.pallas import tpu_sc as plsc

B = 16
ROWS = 896
COLS = 5313
BIN0 = 416
NBIN = 64
NTRK = 32
TSTRIDE = 100
W0 = 384
WREL = BIN0 - W0

_GDN = lax.GatherDimensionNumbers(
    offset_dims=(), collapsed_slice_dims=(0,), start_index_map=(0,)
)


def _perm(v, idx):
    return lax.gather(
        v, idx[:, None], _GDN, (1,), mode=lax.GatherScatterMode.PROMISE_IN_BOUNDS
    )


def _sc_score(y):
    mesh = plsc.VectorSubcoreMesh(core_axis_name="c", subcore_axis_name="s")

    @functools.partial(
        pl.kernel,
        mesh=mesh,
        out_type=jax.ShapeDtypeStruct((NTRK * B,), jnp.float32),
        scratch_types=[
            pltpu.VMEM((B, 128), jnp.float32),
            pltpu.VMEM((B,), jnp.float32),
            pltpu.SemaphoreType.DMA,
        ],
        compiler_params=pltpu.CompilerParams(use_tc_tiling_on_sc=True),
    )
    def k(y_hbm, out_hbm, buf_v, res_v, sem):
        w = lax.axis_index("c") * 16 + lax.axis_index("s")
        pltpu.async_copy(
            y_hbm.at[w * TSTRIDE, pl.ds(0, B), pl.ds(W0, 128)], buf_v, sem
        ).wait()
        lane = lax.iota(jnp.int32, 16)
        xors = [lane ^ k for k in (8, 4, 2, 1)]
        res = jnp.zeros((B,), jnp.float32)
        for b in range(B):
            acc = buf_v[b, pl.ds(WREL, 16)]
            for c in range(1, NBIN // 16):
                acc = acc + buf_v[b, pl.ds(WREL + c * 16, 16)]
            for ix in xors:
                acc = acc + _perm(acc, ix)
            res = jnp.where(lane == b, acc, res)
        res_v[...] = res
        pltpu.sync_copy(res_v, out_hbm.at[pl.ds(w * B, B)])

    return k(y)


def kernel(inputs):
    y = jnp.transpose(inputs, (2, 0, 1))
    out = _sc_score(y)
    return out.reshape(NTRK, B).T


# speedup vs baseline: 13.9430x; 1.0043x over previous
import functools

import jax
import jax.numpy as jnp
from jax import lax
from jax.experimental import pallas as pl
from jax.experimental.pallas import tpu as pltpu
from jax.experimental.pallas import tpu_sc as plsc

B = 16
ROWS = 896
COLS = 5313
BIN0 = 416
NBIN = 64
NTRK = 32
TSTRIDE = 100
W0 = 384
WREL = BIN0 - W0

_GDN = lax.GatherDimensionNumbers(
    offset_dims=(), collapsed_slice_dims=(0,), start_index_map=(0,)
)


def _perm(v, idx):
    return lax.gather(
        v, idx[:, None], _GDN, (1,), mode=lax.GatherScatterMode.PROMISE_IN_BOUNDS
    )


def _sc_score(y):
    mesh = plsc.VectorSubcoreMesh(core_axis_name="c", subcore_axis_name="s")

    @functools.partial(
        pl.kernel,
        mesh=mesh,
        out_type=jax.ShapeDtypeStruct((NTRK * B,), jnp.float32),
        scratch_types=[
            pltpu.VMEM((B, 128), jnp.float32),
            pltpu.VMEM((B,), jnp.float32),
            pltpu.SemaphoreType.DMA,
        ],
        compiler_params=pltpu.CompilerParams(use_tc_tiling_on_sc=True),
    )
    def k(y_hbm, out_hbm, buf_v, res_v, sem):
        w = lax.axis_index("c") * 16 + lax.axis_index("s")
        pltpu.async_copy(
            y_hbm.at[w * TSTRIDE, pl.ds(0, B), pl.ds(W0, 128)], buf_v, sem
        ).wait()
        lane = lax.iota(jnp.int32, 16)
        xors = [lane ^ k for k in (8, 4, 2, 1)]

        def body(b, res):
            acc = buf_v[b, pl.ds(WREL, 16)]
            for c in range(1, NBIN // 16):
                acc = acc + buf_v[b, pl.ds(WREL + c * 16, 16)]
            for ix in xors:
                acc = acc + _perm(acc, ix)
            return jnp.where(lane == b, acc, res)

        res_v[...] = lax.fori_loop(0, B, body, jnp.zeros((B,), jnp.float32))
        pltpu.sync_copy(res_v, out_hbm.at[pl.ds(w * B, B)])

    return k(y)


def kernel(inputs):
    y = jnp.transpose(inputs, (2, 0, 1))
    out = _sc_score(y)
    return out.reshape(NTRK, B).T


# speedup vs baseline: 13.9571x; 1.0010x over previous
import functools

import jax
import jax.numpy as jnp
from jax import lax
from jax.experimental import pallas as pl
from jax.experimental.pallas import tpu as pltpu
from jax.experimental.pallas import tpu_sc as plsc

B = 16
ROWS = 896
COLS = 5313
BIN0 = 416
NBIN = 64
NTRK = 32
TSTRIDE = 100
W0 = 384
WREL = BIN0 - W0

_GDN = lax.GatherDimensionNumbers(
    offset_dims=(), collapsed_slice_dims=(0,), start_index_map=(0,)
)


def _perm(v, idx):
    return lax.gather(
        v, idx[:, None], _GDN, (1,), mode=lax.GatherScatterMode.PROMISE_IN_BOUNDS
    )


def _sc_score(y):
    mesh = plsc.VectorSubcoreMesh(core_axis_name="c", subcore_axis_name="s")

    @functools.partial(
        pl.kernel,
        mesh=mesh,
        out_type=jax.ShapeDtypeStruct((NTRK * B,), jnp.float32),
        scratch_types=[
            pltpu.VMEM((B, 128), jnp.float32),
            pltpu.VMEM((B,), jnp.float32),
            pltpu.SemaphoreType.DMA,
        ],
        compiler_params=pltpu.CompilerParams(
            use_tc_tiling_on_sc=True, skip_device_barrier=True
        ),
    )
    def k(y_hbm, out_hbm, buf_v, res_v, sem):
        w = lax.axis_index("c") * 16 + lax.axis_index("s")
        pltpu.async_copy(
            y_hbm.at[w * TSTRIDE, pl.ds(0, B), pl.ds(W0, 128)], buf_v, sem
        ).wait()
        lane = lax.iota(jnp.int32, 16)
        xors = [lane ^ k for k in (8, 4, 2, 1)]

        def body(b, res):
            acc = buf_v[b, pl.ds(WREL, 16)]
            for c in range(1, NBIN // 16):
                acc = acc + buf_v[b, pl.ds(WREL + c * 16, 16)]
            for ix in xors:
                acc = acc + _perm(acc, ix)
            return jnp.where(lane == b, acc, res)

        res_v[...] = lax.fori_loop(0, B, body, jnp.zeros((B,), jnp.float32))
        pltpu.sync_copy(res_v, out_hbm.at[pl.ds(w * B, B)])

    return k(y)


def kernel(inputs):
    y = jnp.transpose(inputs, (2, 0, 1))
    out = _sc_score(y)
    return out.reshape(NTRK, B).T
